# Initial kernel scaffold; baseline (speedup 1.0000x reference)
import jax
import jax.numpy as jnp
from jax.experimental import pallas as pl


def kernel(x):
    raise NotImplementedError("write your pallas kernel here")


# speedup vs baseline: 1.7505x; 1.7505x over previous
import functools

import jax
import jax.numpy as jnp
from jax import lax
from jax.experimental import pallas as pl
from jax.experimental.pallas import tpu as pltpu

_L = 256
_LATENT = 32
_THRESH = 0.15
_ROWS_PER_BLOCK = 2048


def _body(x_ref, o_ref):
    x = x_ref[...]
    R = x.shape[0]
    lane = lax.broadcasted_iota(jnp.int32, (R, _L), 1)
    cand = jnp.where(x > _THRESH, lane, _L)
    p = jnp.min(cand, axis=1, keepdims=True)
    p = jnp.where(p == _L, 0, p)

    y = x
    for k in (128, 64, 32, 16, 8, 4, 2, 1):
        rolled = jnp.concatenate([y[:, k:], y[:, :k]], axis=1)
        bit = (p & k) != 0
        y = jnp.where(bit, rolled, y)

    fill = jnp.broadcast_to(p.astype(jnp.float32) * (1.0 / _L), (R, _LATENT))
    o_ref[...] = jnp.concatenate([y[:, :_LATENT], fill], axis=1)


@jax.jit
def kernel(x):
    n = x.shape[0]
    out = pl.pallas_call(
        _body,
        grid=(n // _ROWS_PER_BLOCK,),
        in_specs=[pl.BlockSpec((_ROWS_PER_BLOCK, _L), lambda i: (i, 0))],
        out_specs=pl.BlockSpec((_ROWS_PER_BLOCK, 2 * _LATENT), lambda i: (i, 0)),
        out_shape=jax.ShapeDtypeStruct((n, 2 * _LATENT), jnp.float32),
    )(x)
    return out.reshape(n, 1, 2 * _LATENT)
